# baseline (speedup 1.0000x reference)
import functools

import numpy as np
import jax
import jax.numpy as jnp
from jax.experimental import pallas as pl
from jax.experimental.pallas import tpu as pltpu

_NB = 4
_NPOINTS = 16
_CLIP = 20.0


def _perm_tables():
    a = np.zeros([_NPOINTS, _NB], dtype=np.int32)
    for i in range(_NPOINTS):
        a[i, :] = np.array(list(np.binary_repr(i, _NB)), dtype=np.int32)
    c0 = np.zeros([_NPOINTS // 2, _NB], np.int32)
    c1 = np.zeros([_NPOINTS // 2, _NB], np.int32)
    for i in range(_NB):
        c0[:, i] = np.where(a[:, i] == 0)[0]
        c1[:, i] = np.where(a[:, i] == 1)[0]
    return c0.reshape(-1), c1.reshape(-1)


_G0, _G1 = _perm_tables()
_BB = 256
_BIG = np.float32(1e30)


@functools.cache
def _build_tc_kernel(B, K, S):
    KS = K * S
    OUT_J, OUT_I = _NPOINTS // 2, _NB

    p_np = np.zeros((_NPOINTS, OUT_J * OUT_I), np.float32)
    for t in range(OUT_J * OUT_I):
        p_np[_G0[t], t] += 1.0
        p_np[_G1[t], t] -= 1.0

    def body(pi_ref, d_ref, p_ref, out_ref):
        bits3 = 1 << pi_ref[...]
        b4 = bits3[:, :, 0:4] | bits3[:, :, 4:8]
        b2 = b4[:, :, 0:2] | b4[:, :, 2:4]
        bits = b2[:, :, 0] | b2[:, :, 1]
        d = d_ref[...] * 0.5
        ms = [
            jnp.min(jnp.where((bits & (1 << v)) != 0, d, _BIG),
                    axis=1, keepdims=True)
            for v in range(_NPOINTS)
        ]
        m = jnp.concatenate(ms, axis=1)
        llr = jnp.clip(jnp.dot(m, p_ref[...],
                               preferred_element_type=jnp.float32),
                       -_CLIP, _CLIP)
        out_ref[...] = llr.reshape(_BB, OUT_J, OUT_I)

    grid = (B // _BB,)
    call = pl.pallas_call(
        body,
        grid=grid,
        in_specs=[
            pl.BlockSpec((_BB, K, S), lambda i: (i, 0, 0)),
            pl.BlockSpec((_BB, K), lambda i: (i, 0)),
            pl.BlockSpec((_NPOINTS, OUT_J * OUT_I), lambda i: (0, 0)),
        ],
        out_specs=pl.BlockSpec((_BB, OUT_J, OUT_I), lambda i: (i, 0, 0)),
        out_shape=jax.ShapeDtypeStruct((B, OUT_J, OUT_I), jnp.float32),
        compiler_params=pltpu.CompilerParams(
            dimension_semantics=("arbitrary",)),
    )

    def run(path_inds, dists):
        return call(path_inds, dists, jnp.asarray(p_np))

    return run


def kernel(y, r, dists, path_inds, path_syms):
    B, K = dists.shape
    S = path_inds.shape[2]
    return _build_tc_kernel(B, K, S)(path_inds, dists)


# speedup vs baseline: 5.1506x; 5.1506x over previous
import functools

import numpy as np
import jax
import jax.numpy as jnp
from jax.experimental import pallas as pl
from jax.experimental.pallas import tpu as pltpu

_NB = 4
_NPOINTS = 16
_CLIP = 20.0


def _perm_tables():
    a = np.zeros([_NPOINTS, _NB], dtype=np.int32)
    for i in range(_NPOINTS):
        a[i, :] = np.array(list(np.binary_repr(i, _NB)), dtype=np.int32)
    c0 = np.zeros([_NPOINTS // 2, _NB], np.int32)
    c1 = np.zeros([_NPOINTS // 2, _NB], np.int32)
    for i in range(_NB):
        c0[:, i] = np.where(a[:, i] == 0)[0]
        c1[:, i] = np.where(a[:, i] == 1)[0]
    return c0.reshape(-1), c1.reshape(-1)


_G0, _G1 = _perm_tables()
_BB = 512
_BIG = np.float32(1e30)


@functools.cache
def _build_tc_kernel(B, K, S):
    KS = K * S
    OUT_J, OUT_I = _NPOINTS // 2, _NB

    e_np = np.zeros((K, KS), np.float32)
    for k in range(K):
        e_np[k, k * S:(k + 1) * S] = 0.5
    p_np = np.zeros((_NPOINTS, OUT_J * OUT_I), np.float32)
    for t in range(OUT_J * OUT_I):
        p_np[_G0[t], t] += 1.0
        p_np[_G1[t], t] -= 1.0

    def body(pi_ref, d_ref, e_ref, p_ref, out_ref):
        pif = pi_ref[...].reshape(_BB, KS)
        dfull = jnp.dot(d_ref[...], e_ref[...],
                        preferred_element_type=jnp.float32)
        ms = [
            jnp.min(jnp.where(pif == v, dfull, _BIG), axis=1, keepdims=True)
            for v in range(_NPOINTS)
        ]
        m = jnp.concatenate(ms, axis=1)
        llr = jnp.clip(jnp.dot(m, p_ref[...],
                               preferred_element_type=jnp.float32),
                       -_CLIP, _CLIP)
        out_ref[...] = llr.reshape(_BB, OUT_J, OUT_I)

    grid = (B // _BB,)
    call = pl.pallas_call(
        body,
        grid=grid,
        in_specs=[
            pl.BlockSpec((_BB, K, S), lambda i: (i, 0, 0)),
            pl.BlockSpec((_BB, K), lambda i: (i, 0)),
            pl.BlockSpec((K, KS), lambda i: (0, 0)),
            pl.BlockSpec((_NPOINTS, OUT_J * OUT_I), lambda i: (0, 0)),
        ],
        out_specs=pl.BlockSpec((_BB, OUT_J, OUT_I), lambda i: (i, 0, 0)),
        out_shape=jax.ShapeDtypeStruct((B, OUT_J, OUT_I), jnp.float32),
        compiler_params=pltpu.CompilerParams(
            dimension_semantics=("arbitrary",)),
    )

    def run(path_inds, dists):
        return call(path_inds, dists, jnp.asarray(e_np), jnp.asarray(p_np))

    return run


def kernel(y, r, dists, path_inds, path_syms):
    B, K = dists.shape
    S = path_inds.shape[2]
    return _build_tc_kernel(B, K, S)(path_inds, dists)


# speedup vs baseline: 5.7874x; 1.1236x over previous
import functools

import numpy as np
import jax
import jax.numpy as jnp
from jax.experimental import pallas as pl
from jax.experimental.pallas import tpu as pltpu

_NB = 4
_NPOINTS = 16
_CLIP = 20.0


def _perm_tables():
    a = np.zeros([_NPOINTS, _NB], dtype=np.int32)
    for i in range(_NPOINTS):
        a[i, :] = np.array(list(np.binary_repr(i, _NB)), dtype=np.int32)
    c0 = np.zeros([_NPOINTS // 2, _NB], np.int32)
    c1 = np.zeros([_NPOINTS // 2, _NB], np.int32)
    for i in range(_NB):
        c0[:, i] = np.where(a[:, i] == 0)[0]
        c1[:, i] = np.where(a[:, i] == 1)[0]
    return c0.reshape(-1), c1.reshape(-1)


_G0, _G1 = _perm_tables()
_BB = 512
_BIG = np.float32(1e30)


@functools.cache
def _build_tc_kernel(B, K, S):
    KS = K * S
    OUT_J, OUT_I = _NPOINTS // 2, _NB

    e_np = np.zeros((K, KS), np.float32)
    for k in range(K):
        e_np[k, k * S:(k + 1) * S] = 0.5
    p_np = np.zeros((_NPOINTS, OUT_J * OUT_I), np.float32)
    for t in range(OUT_J * OUT_I):
        p_np[_G0[t], t] += 1.0
        p_np[_G1[t], t] -= 1.0

    def body(pi_ref, d_ref, e_ref, p_ref, out_ref):
        pif = pi_ref[...].astype(jnp.bfloat16).reshape(_BB, KS)
        dfull = jnp.dot(d_ref[...], e_ref[...],
                        preferred_element_type=jnp.float32)
        ms = [
            jnp.min(jnp.where(pif == v, dfull, _BIG), axis=1, keepdims=True)
            for v in range(_NPOINTS)
        ]
        m = jnp.concatenate(ms, axis=1)
        llr = jnp.clip(jnp.dot(m, p_ref[...],
                               preferred_element_type=jnp.float32),
                       -_CLIP, _CLIP)
        out_ref[...] = llr.reshape(_BB, OUT_J, OUT_I)

    grid = (B // _BB,)
    call = pl.pallas_call(
        body,
        grid=grid,
        in_specs=[
            pl.BlockSpec((_BB, K, S), lambda i: (i, 0, 0)),
            pl.BlockSpec((_BB, K), lambda i: (i, 0)),
            pl.BlockSpec((K, KS), lambda i: (0, 0)),
            pl.BlockSpec((_NPOINTS, OUT_J * OUT_I), lambda i: (0, 0)),
        ],
        out_specs=pl.BlockSpec((_BB, OUT_J, OUT_I), lambda i: (i, 0, 0)),
        out_shape=jax.ShapeDtypeStruct((B, OUT_J, OUT_I), jnp.float32),
        compiler_params=pltpu.CompilerParams(
            dimension_semantics=("arbitrary",)),
    )

    def run(path_inds, dists):
        return call(path_inds, dists, jnp.asarray(e_np), jnp.asarray(p_np))

    return run


def kernel(y, r, dists, path_inds, path_syms):
    B, K = dists.shape
    S = path_inds.shape[2]
    return _build_tc_kernel(B, K, S)(path_inds, dists)


# speedup vs baseline: 6.0588x; 1.0469x over previous
import functools

import numpy as np
import jax
import jax.numpy as jnp
from jax.experimental import pallas as pl
from jax.experimental.pallas import tpu as pltpu

_NB = 4
_NPOINTS = 16
_CLIP = 20.0


def _perm_tables():
    a = np.zeros([_NPOINTS, _NB], dtype=np.int32)
    for i in range(_NPOINTS):
        a[i, :] = np.array(list(np.binary_repr(i, _NB)), dtype=np.int32)
    c0 = np.zeros([_NPOINTS // 2, _NB], np.int32)
    c1 = np.zeros([_NPOINTS // 2, _NB], np.int32)
    for i in range(_NB):
        c0[:, i] = np.where(a[:, i] == 0)[0]
        c1[:, i] = np.where(a[:, i] == 1)[0]
    return c0.reshape(-1), c1.reshape(-1)


_G0, _G1 = _perm_tables()
_BB = 512
_BIG = np.float32(1e30)


@functools.cache
def _build_tc_kernel(B, K, S):
    KS = K * S
    OUT_J, OUT_I = _NPOINTS // 2, _NB

    e_np = np.zeros((K, KS), np.float32)
    for k in range(K):
        e_np[k, k * S:(k + 1) * S] = 0.5
    p_np = np.zeros((_NPOINTS, OUT_J * OUT_I), np.float32)
    for t in range(OUT_J * OUT_I):
        p_np[_G0[t], t] += 1.0
        p_np[_G1[t], t] -= 1.0

    def body(pi_ref, d_ref, e_ref, p_ref, out_ref):
        pif = pi_ref[...].astype(jnp.bfloat16).reshape(_BB, KS)
        dfull = jnp.dot(d_ref[...], e_ref[...],
                        preferred_element_type=jnp.float32
                        ).astype(jnp.bfloat16)
        big = jnp.bfloat16(_BIG)
        ms = [
            jnp.min(jnp.where(pif == v, dfull, big), axis=1, keepdims=True)
            for v in range(_NPOINTS)
        ]
        m = jnp.concatenate(ms, axis=1)
        llr = jnp.clip(jnp.dot(m, p_ref[...].astype(jnp.bfloat16),
                               preferred_element_type=jnp.float32),
                       -_CLIP, _CLIP)
        out_ref[...] = llr.reshape(_BB, OUT_J, OUT_I)

    grid = (B // _BB,)
    call = pl.pallas_call(
        body,
        grid=grid,
        in_specs=[
            pl.BlockSpec((_BB, K, S), lambda i: (i, 0, 0)),
            pl.BlockSpec((_BB, K), lambda i: (i, 0)),
            pl.BlockSpec((K, KS), lambda i: (0, 0)),
            pl.BlockSpec((_NPOINTS, OUT_J * OUT_I), lambda i: (0, 0)),
        ],
        out_specs=pl.BlockSpec((_BB, OUT_J, OUT_I), lambda i: (i, 0, 0)),
        out_shape=jax.ShapeDtypeStruct((B, OUT_J, OUT_I), jnp.float32),
        compiler_params=pltpu.CompilerParams(
            dimension_semantics=("arbitrary",)),
    )

    def run(path_inds, dists):
        return call(path_inds, dists, jnp.asarray(e_np), jnp.asarray(p_np))

    return run


def kernel(y, r, dists, path_inds, path_syms):
    B, K = dists.shape
    S = path_inds.shape[2]
    return _build_tc_kernel(B, K, S)(path_inds, dists)
